# baseline (speedup 1.0000x reference)
import functools

import numpy as np
import jax
import jax.numpy as jnp
from jax import lax
from jax.experimental import pallas as pl
from jax.experimental.pallas import tpu as pltpu
from jax.experimental.pallas import tpu_sc as plsc

D = H = W = 128
NC = 2
NS = 16
LANES = 16
NW = NC * NS
SLAB = D // NW
CHUNK = 64
NROW = 16
RADIUS = 5
WIN = 2 * RADIUS + 1
NJ = 8
TBLK = 512
INV64 = 1.0 / (0.5 * D)


def _preprocess_body(p0, p1, p2, s0, s1, s2, q0, q1, q2, q3, den, out):
    half = jnp.float32(0.5 * D)
    c0 = (p0[...] + 1.0) * half - 0.5
    c1 = (p1[...] + 1.0) * half - 0.5
    c2 = (p2[...] + 1.0) * half - 0.5
    a, b, c, d = q0[...], q1[...], q2[...], q3[...]
    nrm = jnp.sqrt(a * a + b * b + c * c + d * d) + 1e-8
    a, b, c, d = a / nrm, b / nrm, c / nrm, d / nrm
    r00 = 1 - 2 * (c * c + d * d)
    r01 = 2 * (b * c - a * d)
    r02 = 2 * (b * d + a * c)
    r10 = 2 * (b * c + a * d)
    r11 = 1 - 2 * (b * b + d * d)
    r12 = 2 * (c * d - a * b)
    r20 = 2 * (b * d - a * c)
    r21 = 2 * (c * d + a * b)
    r22 = 1 - 2 * (b * b + c * c)
    se0 = s0[...] + 1e-8
    se1 = s1[...] + 1e-8
    se2 = s2[...] + 1e-8
    i0 = 1.0 / se0
    i1 = 1.0 / se1
    i2 = 1.0 / se2
    l00, l01, l02 = r00 * i0, r01 * i1, r02 * i2
    l10, l11, l12 = r10 * i0, r11 * i1, r12 * i2
    l20, l21, l22 = r20 * i0, r21 * i1, r22 * i2
    m00 = l00 * l00 + l01 * l01 + l02 * l02
    m11 = l10 * l10 + l11 * l11 + l12 * l12
    m22 = l20 * l20 + l21 * l21 + l22 * l22
    m01 = l00 * l10 + l01 * l11 + l02 * l12
    m02 = l00 * l20 + l01 * l21 + l02 * l22
    m12 = l10 * l20 + l11 * l21 + l12 * l22
    e0sq, e1sq, e2sq = se0 * se0, se1 * se1, se2 * se2
    v0 = r00 * r00 * e0sq + r01 * r01 * e1sq + r02 * r02 * e2sq
    v1 = r10 * r10 * e0sq + r11 * r11 * e1sq + r12 * r12 * e2sq
    v2 = r20 * r20 * e0sq + r21 * r21 * e1sq + r22 * r22 * e2sq
    scale3 = jnp.float32(3.0 * 0.5 * D)
    cap = jnp.float32(RADIUS)
    rz = jnp.minimum(jnp.floor(jnp.sqrt(v0) * scale3 + 0.502), cap)
    ry = jnp.minimum(jnp.floor(jnp.sqrt(v1) * scale3 + 0.502), cap)
    rx = jnp.minimum(jnp.floor(jnp.sqrt(v2) * scale3 + 0.502), cap)
    out[0] = c0
    out[1] = c1
    out[2] = c2
    out[3] = m00 * jnp.float32(-0.5)
    out[4] = m11
    out[5] = m22
    out[6] = m01
    out[7] = m02
    out[8] = m12
    out[9] = ry
    out[10] = jnp.round(c0)
    out[11] = jnp.round(c1)
    out[12] = jnp.round(c2)
    out[13] = jnp.log(den[...])
    out[14] = rz
    out[15] = rx


def _tables_body(pref, oq, og, oi):
    p = pref[...]
    c1 = p[1][:, None]
    c2 = p[2][:, None]
    m11 = p[4][:, None]
    m22 = p[5][:, None]
    m01 = p[6][:, None]
    m02 = p[7][:, None]
    m12 = p[8][:, None]
    r1 = p[11][:, None]
    r2 = p[12][:, None]
    kf = lax.broadcasted_iota(jnp.int32, (TBLK, 128), 1).astype(jnp.float32)
    dyk = jnp.floor((kf + 0.5) * jnp.float32(1.0 / WIN)) - RADIUS
    dxk = kf - (dyk + RADIUS) * WIN - RADIUS
    y = r1 + dyk
    x = r2 + dxk
    inv = jnp.float32(INV64)
    d1 = (y - c1) * inv
    d2 = (x - c2) * inv
    qy = (m11 * d1 * d1 + m22 * d2 * d2 +
          jnp.float32(2.0) * m12 * d1 * d2) * jnp.float32(-0.5)
    gy = -(m01 * d1 + m02 * d2)
    fw = jnp.float32(W)
    valid = ((y >= 0) & (y <= fw - 1) & (x >= 0) & (x <= fw - 1)
             & (kf <= jnp.float32(WIN * WIN - 1)))
    lden = p[13][:, None]
    oq[...] = jnp.where(valid, qy + lden, jnp.float32(-1e30))
    og[...] = gy
    oi[...] = jnp.where(valid, y * fw + x, 0.0).astype(jnp.int32)


def _sc_body(ranges_hbm, params_hbm, qy_hbm, gy_hbm, iyx_hbm, out_hbm,
             rbuf, pbuf, qbuf, gbuf, ibuf, vol, sem0, sem1):
    wid = lax.axis_index("s") * NC + lax.axis_index("c")
    z_lo = wid * SLAB
    z_hi = z_lo + SLAB
    zeros16 = jnp.zeros((LANES,), jnp.float32)

    def _zero(i, _):
        vol[pl.ds(i * LANES, LANES)] = zeros16
        return ()
    lax.fori_loop(0, (SLAB * H * W) // LANES, _zero, ())

    pltpu.sync_copy(ranges_hbm, rbuf)
    rv = rbuf[pl.ds(wid * 16, 16)]
    start = rv[0]
    end = rv[1]
    nch = (end - start + (CHUNK - 1)) // CHUNK
    PRE = CHUNK * NROW
    TBL = CHUNK * 128
    sems = (sem0, sem1)

    def start_dmas(ci, slot):
        gbase = start + ci * CHUNK
        sem = sems[slot]
        pltpu.async_copy(params_hbm.at[pl.ds(gbase * NROW, PRE)],
                         pbuf.at[pl.ds(slot * PRE, PRE)], sem)
        pltpu.async_copy(qy_hbm.at[pl.ds(gbase * 128, TBL)],
                         qbuf.at[pl.ds(slot * TBL, TBL)], sem)
        pltpu.async_copy(gy_hbm.at[pl.ds(gbase * 128, TBL)],
                         gbuf.at[pl.ds(slot * TBL, TBL)], sem)
        pltpu.async_copy(iyx_hbm.at[pl.ds(gbase * 128, TBL)],
                         ibuf.at[pl.ds(slot * TBL, TBL)], sem)

    def wait_dmas(slot):
        sem = sems[slot]
        pltpu.make_async_copy(params_hbm.at[pl.ds(0, PRE)],
                              pbuf.at[pl.ds(slot * PRE, PRE)], sem).wait()
        pltpu.make_async_copy(qy_hbm.at[pl.ds(0, TBL)],
                              qbuf.at[pl.ds(slot * TBL, TBL)], sem).wait()
        pltpu.make_async_copy(gy_hbm.at[pl.ds(0, TBL)],
                              gbuf.at[pl.ds(slot * TBL, TBL)], sem).wait()
        pltpu.make_async_copy(iyx_hbm.at[pl.ds(0, TBL)],
                              ibuf.at[pl.ds(slot * TBL, TBL)], sem).wait()

    inv = jnp.float32(INV64)

    def process(ci, slot):
        gbase = start + ci * CHUNK
        cnt = jnp.minimum(CHUNK, end - gbase)
        po = slot * PRE
        to = slot * TBL

        def per_gaussian(gl, _):
            pv = pbuf[pl.ds(po + gl * NROW, NROW)]
            c0s = pv[0]
            nm00 = pv[3]
            r0i = pv[10].astype(jnp.int32)
            thv = jnp.full((LANES,), pv[13] + jnp.float32(-4.5),
                           jnp.float32)
            tbase = to + gl * 128
            lo = jnp.maximum(r0i - RADIUS, z_lo)
            hi = jnp.minimum(r0i + RADIUS + 1, z_hi)

            def per_plane(z, _):
                d0s = (z.astype(jnp.float32) - c0s) * inv
                t0s = nm00 * d0s * d0s
                d0 = jnp.full((LANES,), d0s, jnp.float32)
                t0 = jnp.full((LANES,), t0s, jnp.float32)
                zoff = jnp.full((LANES,), (z - z_lo) * (H * W), jnp.int32)

                def group(j):
                    off = tbase + j * LANES
                    e = t0 + qbuf[pl.ds(off, LANES)] \
                        + d0 * gbuf[pl.ds(off, LANES)]
                    w = jnp.exp(e)
                    msk = e >= thv
                    plsc.addupdate_scatter(
                        vol, [zoff + ibuf[pl.ds(off, LANES)]], w, mask=msk)

                for j in range(NJ):
                    group(j)
                return ()

            lax.fori_loop(lo, hi, per_plane, ())
            return ()

        lax.fori_loop(0, cnt, per_gaussian, ())

    @pl.when(nch > 0)
    def _():
        start_dmas(0, 0)

    @pl.when(nch > 1)
    def _():
        start_dmas(1, 1)

    def pair_body(pi, _):
        for slot in range(2):
            ci = pi * 2 + slot

            @pl.when(ci < nch)
            def _():
                wait_dmas(slot)
                process(ci, slot)

                @pl.when(ci + 2 < nch)
                def _():
                    start_dmas(ci + 2, slot)
        return ()

    lax.fori_loop(0, (nch + 1) // 2, pair_body, ())
    pltpu.sync_copy(vol, out_hbm.at[pl.ds(wid * (SLAB * H * W), SLAB * H * W)])


@jax.jit
def _voxelize(positions, scales, rotations, density):
    n = positions.shape[0]
    npad = ((n + 127) // 128) * 128
    rows = npad // 128
    npad2 = ((npad + CHUNK + TBLK - 1) // TBLK) * TBLK

    def pad(x, fill):
        return jnp.concatenate(
            [x, jnp.full((npad - n,) + x.shape[1:], fill, x.dtype)], axis=0)

    p = pad(positions, 1.0e5)
    s = pad(scales, 1.0)
    q = pad(rotations, 1.0)
    dn = pad(density, 0.0)

    def col(x, i):
        return x[:, i].reshape(rows, 128)

    ins = [col(p, 0), col(p, 1), col(p, 2),
           col(s, 0), col(s, 1), col(s, 2),
           col(q, 0), col(q, 1), col(q, 2), col(q, 3),
           dn.reshape(rows, 128)]

    params = pl.pallas_call(
        _preprocess_body,
        out_shape=jax.ShapeDtypeStruct((NROW, rows, 128), jnp.float32),
    )(*ins)
    params = params.reshape(NROW, npad)

    order = jnp.argsort(params[10])
    ps = params[:, order]
    ps = jnp.pad(ps, ((0, 0), (0, npad2 - npad)), constant_values=1.0e9)
    r0s = ps[10]
    zlos = (jnp.arange(NW, dtype=jnp.int32) * SLAB).astype(jnp.float32)
    starts = jnp.searchsorted(r0s, zlos - RADIUS, side="left")
    ends = jnp.searchsorted(r0s, zlos + SLAB - 1 + RADIUS, side="right")
    ranges = jnp.zeros((NW, 16), jnp.int32)
    ranges = ranges.at[:, 0].set(starts.astype(jnp.int32))
    ranges = ranges.at[:, 1].set(ends.astype(jnp.int32))

    qy, gy, iyx = pl.pallas_call(
        _tables_body,
        grid=(npad2 // TBLK,),
        in_specs=[pl.BlockSpec((NROW, TBLK), lambda i: (0, i))],
        out_specs=[pl.BlockSpec((TBLK, 128), lambda i: (i, 0)),
                   pl.BlockSpec((TBLK, 128), lambda i: (i, 0)),
                   pl.BlockSpec((TBLK, 128), lambda i: (i, 0))],
        out_shape=[jax.ShapeDtypeStruct((npad2, 128), jnp.float32),
                   jax.ShapeDtypeStruct((npad2, 128), jnp.float32),
                   jax.ShapeDtypeStruct((npad2, 128), jnp.int32)],
    )(ps)

    mesh = plsc.VectorSubcoreMesh(
        core_axis_name="c", subcore_axis_name="s", num_cores=NC,
        num_subcores=NS)
    vol_flat = pl.kernel(
        _sc_body,
        out_type=jax.ShapeDtypeStruct((D * H * W,), jnp.float32),
        mesh=mesh,
        compiler_params=pltpu.CompilerParams(needs_layout_passes=False),
        scratch_types=[
            pltpu.VMEM((NW * 16,), jnp.int32),
            pltpu.VMEM((2 * CHUNK * NROW,), jnp.float32),
            pltpu.VMEM((2 * CHUNK * 128,), jnp.float32),
            pltpu.VMEM((2 * CHUNK * 128,), jnp.float32),
            pltpu.VMEM((2 * CHUNK * 128,), jnp.int32),
            pltpu.VMEM((SLAB * H * W,), jnp.float32),
            pltpu.SemaphoreType.DMA,
            pltpu.SemaphoreType.DMA,
        ],
    )(ranges.reshape(-1), ps.T.reshape(-1), qy.reshape(-1),
      gy.reshape(-1), iyx.reshape(-1))
    return vol_flat.reshape(D, H, W)


def kernel(positions, scales, rotations, density, chunk_size=2048):
    del chunk_size
    return _voxelize(positions, scales, rotations, density)


# speedup vs baseline: 2.9049x; 2.9049x over previous
import functools

import numpy as np
import jax
import jax.numpy as jnp
from jax import lax
from jax.experimental import pallas as pl
from jax.experimental.pallas import tpu as pltpu
from jax.experimental.pallas import tpu_sc as plsc

D = H = W = 128
NC = 2
NS = 16
LANES = 16
NW = NC * NS
SLAB = D // NW
CHUNK = 64
NROW = 16
RADIUS = 5
WIN = 2 * RADIUS + 1
NJ = 8
TBLK = 512
INV64 = 1.0 / (0.5 * D)


def _preprocess_body(p0, p1, p2, s0, s1, s2, q0, q1, q2, q3, den, out):
    half = jnp.float32(0.5 * D)
    c0 = (p0[...] + 1.0) * half - 0.5
    c1 = (p1[...] + 1.0) * half - 0.5
    c2 = (p2[...] + 1.0) * half - 0.5
    a, b, c, d = q0[...], q1[...], q2[...], q3[...]
    nrm = jnp.sqrt(a * a + b * b + c * c + d * d) + 1e-8
    a, b, c, d = a / nrm, b / nrm, c / nrm, d / nrm
    r00 = 1 - 2 * (c * c + d * d)
    r01 = 2 * (b * c - a * d)
    r02 = 2 * (b * d + a * c)
    r10 = 2 * (b * c + a * d)
    r11 = 1 - 2 * (b * b + d * d)
    r12 = 2 * (c * d - a * b)
    r20 = 2 * (b * d - a * c)
    r21 = 2 * (c * d + a * b)
    r22 = 1 - 2 * (b * b + c * c)
    se0 = s0[...] + 1e-8
    se1 = s1[...] + 1e-8
    se2 = s2[...] + 1e-8
    i0 = 1.0 / se0
    i1 = 1.0 / se1
    i2 = 1.0 / se2
    l00, l01, l02 = r00 * i0, r01 * i1, r02 * i2
    l10, l11, l12 = r10 * i0, r11 * i1, r12 * i2
    l20, l21, l22 = r20 * i0, r21 * i1, r22 * i2
    m00 = l00 * l00 + l01 * l01 + l02 * l02
    m11 = l10 * l10 + l11 * l11 + l12 * l12
    m22 = l20 * l20 + l21 * l21 + l22 * l22
    m01 = l00 * l10 + l01 * l11 + l02 * l12
    m02 = l00 * l20 + l01 * l21 + l02 * l22
    m12 = l10 * l20 + l11 * l21 + l12 * l22
    e0sq, e1sq, e2sq = se0 * se0, se1 * se1, se2 * se2
    v0 = r00 * r00 * e0sq + r01 * r01 * e1sq + r02 * r02 * e2sq
    v1 = r10 * r10 * e0sq + r11 * r11 * e1sq + r12 * r12 * e2sq
    v2 = r20 * r20 * e0sq + r21 * r21 * e1sq + r22 * r22 * e2sq
    scale3 = jnp.float32(3.0 * 0.5 * D)
    cap = jnp.float32(RADIUS)
    rz = jnp.minimum(jnp.floor(jnp.sqrt(v0) * scale3 + 0.502), cap)
    ry = jnp.minimum(jnp.floor(jnp.sqrt(v1) * scale3 + 0.502), cap)
    rx = jnp.minimum(jnp.floor(jnp.sqrt(v2) * scale3 + 0.502), cap)
    out[0] = c0
    out[1] = c1
    out[2] = c2
    out[3] = m00 * jnp.float32(-0.5)
    out[4] = m11
    out[5] = m22
    out[6] = m01
    out[7] = m02
    out[8] = m12
    out[9] = ry
    out[10] = jnp.round(c0)
    out[11] = jnp.round(c1)
    out[12] = jnp.round(c2)
    out[13] = jnp.log(den[...])
    out[14] = rz
    out[15] = rx


def _tables_body(pref, oq, og, oi):
    p = pref[...]
    c1 = p[1][:, None]
    c2 = p[2][:, None]
    m11 = p[4][:, None]
    m22 = p[5][:, None]
    m01 = p[6][:, None]
    m02 = p[7][:, None]
    m12 = p[8][:, None]
    r1 = p[11][:, None]
    r2 = p[12][:, None]
    kf = lax.broadcasted_iota(jnp.int32, (TBLK, 128), 1).astype(jnp.float32)
    dyk = jnp.floor((kf + 0.5) * jnp.float32(1.0 / WIN)) - RADIUS
    dxk = kf - (dyk + RADIUS) * WIN - RADIUS
    y = r1 + dyk
    x = r2 + dxk
    inv = jnp.float32(INV64)
    d1 = (y - c1) * inv
    d2 = (x - c2) * inv
    qy = (m11 * d1 * d1 + m22 * d2 * d2 +
          jnp.float32(2.0) * m12 * d1 * d2) * jnp.float32(-0.5)
    gy = -(m01 * d1 + m02 * d2)
    fw = jnp.float32(W)
    valid = ((y >= 0) & (y <= fw - 1) & (x >= 0) & (x <= fw - 1)
             & (kf <= jnp.float32(WIN * WIN - 1)))
    lden = p[13][:, None]
    oq[...] = jnp.where(valid, qy + lden, jnp.float32(-1e30))
    og[...] = gy
    oi[...] = jnp.where(valid, y * fw + x, 0.0).astype(jnp.int32)


def _sc_body(ranges_hbm, params_hbm, qy_hbm, gy_hbm, iyx_hbm, out_hbm,
             rbuf, pbuf, qbuf, gbuf, ibuf, vol, sem0, sem1):
    wid = lax.axis_index("s") * NC + lax.axis_index("c")
    z_lo = wid * SLAB
    z_hi = z_lo + SLAB
    zeros16 = jnp.zeros((LANES,), jnp.float32)

    def _zero(i, _):
        vol[pl.ds(i * LANES, LANES)] = zeros16
        return ()
    lax.fori_loop(0, (SLAB * H * W) // LANES, _zero, ())

    pltpu.sync_copy(ranges_hbm, rbuf)
    rv = rbuf[pl.ds(wid * 16, 16)]
    start = rv[0]
    end = rv[1]
    nch = (end - start + (CHUNK - 1)) // CHUNK
    PRE = CHUNK * NROW
    TBL = CHUNK * 128
    sems = (sem0, sem1)

    def start_dmas(ci, slot):
        gbase = start + ci * CHUNK
        sem = sems[slot]
        pltpu.async_copy(params_hbm.at[pl.ds(gbase * NROW, PRE)],
                         pbuf.at[pl.ds(slot * PRE, PRE)], sem)
        pltpu.async_copy(qy_hbm.at[pl.ds(gbase * 128, TBL)],
                         qbuf.at[pl.ds(slot * TBL, TBL)], sem)
        pltpu.async_copy(gy_hbm.at[pl.ds(gbase * 128, TBL)],
                         gbuf.at[pl.ds(slot * TBL, TBL)], sem)
        pltpu.async_copy(iyx_hbm.at[pl.ds(gbase * 128, TBL)],
                         ibuf.at[pl.ds(slot * TBL, TBL)], sem)

    def wait_dmas(slot):
        sem = sems[slot]
        pltpu.make_async_copy(params_hbm.at[pl.ds(0, PRE)],
                              pbuf.at[pl.ds(slot * PRE, PRE)], sem).wait()
        pltpu.make_async_copy(qy_hbm.at[pl.ds(0, TBL)],
                              qbuf.at[pl.ds(slot * TBL, TBL)], sem).wait()
        pltpu.make_async_copy(gy_hbm.at[pl.ds(0, TBL)],
                              gbuf.at[pl.ds(slot * TBL, TBL)], sem).wait()
        pltpu.make_async_copy(iyx_hbm.at[pl.ds(0, TBL)],
                              ibuf.at[pl.ds(slot * TBL, TBL)], sem).wait()

    inv = jnp.float32(INV64)

    def process(ci, slot):
        gbase = start + ci * CHUNK
        cnt = jnp.minimum(CHUNK, end - gbase)
        po = slot * PRE
        to = slot * TBL

        def per_gaussian(gl, _):
            pv = pbuf[pl.ds(po + gl * NROW, NROW)]
            c0s = pv[0]
            nm00 = pv[3]
            r0i = pv[10].astype(jnp.int32)
            thv = jnp.full((LANES,), pv[13] + jnp.float32(-4.5),
                           jnp.float32)
            qv = [qbuf[pl.ds(to + gl * 128 + j * LANES, LANES)]
                  for j in range(NJ)]
            gv = [gbuf[pl.ds(to + gl * 128 + j * LANES, LANES)]
                  for j in range(NJ)]
            iv = [ibuf[pl.ds(to + gl * 128 + j * LANES, LANES)]
                  for j in range(NJ)]
            lo = jnp.maximum(r0i - RADIUS, z_lo)
            hi = jnp.minimum(r0i + RADIUS + 1, z_hi)

            def per_plane(z, _):
                d0s = (z.astype(jnp.float32) - c0s) * inv
                t0s = nm00 * d0s * d0s
                d0 = jnp.full((LANES,), d0s, jnp.float32)
                t0 = jnp.full((LANES,), t0s, jnp.float32)
                zoff = jnp.full((LANES,), (z - z_lo) * (H * W), jnp.int32)
                for j in range(NJ):
                    e = t0 + qv[j] + d0 * gv[j]
                    w = jnp.exp(e)
                    msk = e >= thv
                    plsc.addupdate_scatter(vol, [zoff + iv[j]], w, mask=msk)
                return ()

            lax.fori_loop(lo, hi, per_plane, ())
            return ()

        lax.fori_loop(0, cnt, per_gaussian, ())

    @pl.when(nch > 0)
    def _():
        start_dmas(0, 0)

    @pl.when(nch > 1)
    def _():
        start_dmas(1, 1)

    def pair_body(pi, _):
        for slot in range(2):
            ci = pi * 2 + slot

            @pl.when(ci < nch)
            def _():
                wait_dmas(slot)
                process(ci, slot)

                @pl.when(ci + 2 < nch)
                def _():
                    start_dmas(ci + 2, slot)
        return ()

    lax.fori_loop(0, (nch + 1) // 2, pair_body, ())
    pltpu.sync_copy(vol, out_hbm.at[pl.ds(wid * (SLAB * H * W), SLAB * H * W)])


@jax.jit
def _voxelize(positions, scales, rotations, density):
    n = positions.shape[0]
    npad = ((n + 127) // 128) * 128
    rows = npad // 128
    npad2 = ((npad + CHUNK + TBLK - 1) // TBLK) * TBLK

    def pad(x, fill):
        return jnp.concatenate(
            [x, jnp.full((npad - n,) + x.shape[1:], fill, x.dtype)], axis=0)

    p = pad(positions, 1.0e5)
    s = pad(scales, 1.0)
    q = pad(rotations, 1.0)
    dn = pad(density, 0.0)

    def col(x, i):
        return x[:, i].reshape(rows, 128)

    ins = [col(p, 0), col(p, 1), col(p, 2),
           col(s, 0), col(s, 1), col(s, 2),
           col(q, 0), col(q, 1), col(q, 2), col(q, 3),
           dn.reshape(rows, 128)]

    params = pl.pallas_call(
        _preprocess_body,
        out_shape=jax.ShapeDtypeStruct((NROW, rows, 128), jnp.float32),
    )(*ins)
    params = params.reshape(NROW, npad)

    order = jnp.argsort(params[10])
    ps = params[:, order]
    ps = jnp.pad(ps, ((0, 0), (0, npad2 - npad)), constant_values=1.0e9)
    r0s = ps[10]
    zlos = (jnp.arange(NW, dtype=jnp.int32) * SLAB).astype(jnp.float32)
    starts = jnp.searchsorted(r0s, zlos - RADIUS, side="left")
    ends = jnp.searchsorted(r0s, zlos + SLAB - 1 + RADIUS, side="right")
    ranges = jnp.zeros((NW, 16), jnp.int32)
    ranges = ranges.at[:, 0].set(starts.astype(jnp.int32))
    ranges = ranges.at[:, 1].set(ends.astype(jnp.int32))

    qy, gy, iyx = pl.pallas_call(
        _tables_body,
        grid=(npad2 // TBLK,),
        in_specs=[pl.BlockSpec((NROW, TBLK), lambda i: (0, i))],
        out_specs=[pl.BlockSpec((TBLK, 128), lambda i: (i, 0)),
                   pl.BlockSpec((TBLK, 128), lambda i: (i, 0)),
                   pl.BlockSpec((TBLK, 128), lambda i: (i, 0))],
        out_shape=[jax.ShapeDtypeStruct((npad2, 128), jnp.float32),
                   jax.ShapeDtypeStruct((npad2, 128), jnp.float32),
                   jax.ShapeDtypeStruct((npad2, 128), jnp.int32)],
    )(ps)

    mesh = plsc.VectorSubcoreMesh(
        core_axis_name="c", subcore_axis_name="s", num_cores=NC,
        num_subcores=NS)
    vol_flat = pl.kernel(
        _sc_body,
        out_type=jax.ShapeDtypeStruct((D * H * W,), jnp.float32),
        mesh=mesh,
        compiler_params=pltpu.CompilerParams(needs_layout_passes=False),
        scratch_types=[
            pltpu.VMEM((NW * 16,), jnp.int32),
            pltpu.VMEM((2 * CHUNK * NROW,), jnp.float32),
            pltpu.VMEM((2 * CHUNK * 128,), jnp.float32),
            pltpu.VMEM((2 * CHUNK * 128,), jnp.float32),
            pltpu.VMEM((2 * CHUNK * 128,), jnp.int32),
            pltpu.VMEM((SLAB * H * W,), jnp.float32),
            pltpu.SemaphoreType.DMA,
            pltpu.SemaphoreType.DMA,
        ],
    )(ranges.reshape(-1), ps.T.reshape(-1), qy.reshape(-1),
      gy.reshape(-1), iyx.reshape(-1))
    return vol_flat.reshape(D, H, W)


def kernel(positions, scales, rotations, density, chunk_size=2048):
    del chunk_size
    return _voxelize(positions, scales, rotations, density)


# speedup vs baseline: 3.3101x; 1.1395x over previous
import functools

import numpy as np
import jax
import jax.numpy as jnp
from jax import lax
from jax.experimental import pallas as pl
from jax.experimental.pallas import tpu as pltpu
from jax.experimental.pallas import tpu_sc as plsc

D = H = W = 128
NC = 2
NS = 16
LANES = 16
NW = NC * NS
SLAB = D // NW
CHUNK = 64
NROW = 16
RADIUS = 5
WIN = 2 * RADIUS + 1
NJ = 8
TBLK = 512
INV64 = 1.0 / (0.5 * D)


def _preprocess_body(p0, p1, p2, s0, s1, s2, q0, q1, q2, q3, den, out):
    half = jnp.float32(0.5 * D)
    c0 = (p0[...] + 1.0) * half - 0.5
    c1 = (p1[...] + 1.0) * half - 0.5
    c2 = (p2[...] + 1.0) * half - 0.5
    a, b, c, d = q0[...], q1[...], q2[...], q3[...]
    nrm = jnp.sqrt(a * a + b * b + c * c + d * d) + 1e-8
    a, b, c, d = a / nrm, b / nrm, c / nrm, d / nrm
    r00 = 1 - 2 * (c * c + d * d)
    r01 = 2 * (b * c - a * d)
    r02 = 2 * (b * d + a * c)
    r10 = 2 * (b * c + a * d)
    r11 = 1 - 2 * (b * b + d * d)
    r12 = 2 * (c * d - a * b)
    r20 = 2 * (b * d - a * c)
    r21 = 2 * (c * d + a * b)
    r22 = 1 - 2 * (b * b + c * c)
    se0 = s0[...] + 1e-8
    se1 = s1[...] + 1e-8
    se2 = s2[...] + 1e-8
    i0 = 1.0 / se0
    i1 = 1.0 / se1
    i2 = 1.0 / se2
    l00, l01, l02 = r00 * i0, r01 * i1, r02 * i2
    l10, l11, l12 = r10 * i0, r11 * i1, r12 * i2
    l20, l21, l22 = r20 * i0, r21 * i1, r22 * i2
    m00 = l00 * l00 + l01 * l01 + l02 * l02
    m11 = l10 * l10 + l11 * l11 + l12 * l12
    m22 = l20 * l20 + l21 * l21 + l22 * l22
    m01 = l00 * l10 + l01 * l11 + l02 * l12
    m02 = l00 * l20 + l01 * l21 + l02 * l22
    m12 = l10 * l20 + l11 * l21 + l12 * l22
    e0sq, e1sq, e2sq = se0 * se0, se1 * se1, se2 * se2
    v0 = r00 * r00 * e0sq + r01 * r01 * e1sq + r02 * r02 * e2sq
    v1 = r10 * r10 * e0sq + r11 * r11 * e1sq + r12 * r12 * e2sq
    v2 = r20 * r20 * e0sq + r21 * r21 * e1sq + r22 * r22 * e2sq
    scale3 = jnp.float32(3.0 * 0.5 * D)
    cap = jnp.float32(RADIUS)
    rz = jnp.minimum(jnp.floor(jnp.sqrt(v0) * scale3 + 0.502), cap)
    ry = jnp.minimum(jnp.floor(jnp.sqrt(v1) * scale3 + 0.502), cap)
    rx = jnp.minimum(jnp.floor(jnp.sqrt(v2) * scale3 + 0.502), cap)
    out[0] = c0
    out[1] = c1
    out[2] = c2
    out[3] = m00 * jnp.float32(-0.5)
    out[4] = m11
    out[5] = m22
    out[6] = m01
    out[7] = m02
    out[8] = m12
    out[9] = ry
    out[10] = jnp.round(c0)
    out[11] = jnp.round(c1)
    out[12] = jnp.round(c2)
    out[13] = jnp.log(den[...])
    out[14] = rz
    out[15] = rx


def _tables_body(pref, oq, og, oi):
    p = pref[...]
    c1 = p[1][:, None]
    c2 = p[2][:, None]
    m11 = p[4][:, None]
    m22 = p[5][:, None]
    m01 = p[6][:, None]
    m02 = p[7][:, None]
    m12 = p[8][:, None]
    r1 = p[11][:, None]
    r2 = p[12][:, None]
    kf = lax.broadcasted_iota(jnp.int32, (TBLK, 128), 1).astype(jnp.float32)
    dyk = jnp.floor((kf + 0.5) * jnp.float32(1.0 / WIN)) - RADIUS
    dxk = kf - (dyk + RADIUS) * WIN - RADIUS
    y = r1 + dyk
    x = r2 + dxk
    inv = jnp.float32(INV64)
    d1 = (y - c1) * inv
    d2 = (x - c2) * inv
    qy = (m11 * d1 * d1 + m22 * d2 * d2 +
          jnp.float32(2.0) * m12 * d1 * d2) * jnp.float32(-0.5)
    gy = -(m01 * d1 + m02 * d2)
    fw = jnp.float32(W)
    valid = ((y >= 0) & (y <= fw - 1) & (x >= 0) & (x <= fw - 1)
             & (kf <= jnp.float32(WIN * WIN - 1)))
    lden = p[13][:, None]
    oq[...] = jnp.where(valid, qy + lden, jnp.float32(-1e30))
    og[...] = gy
    oi[...] = jnp.where(valid, y * fw + x, 0.0).astype(jnp.int32)


def _sc_body(ranges_hbm, params_hbm, qy_hbm, gy_hbm, iyx_hbm, out_hbm,
             rbuf, pbuf, qbuf, gbuf, ibuf, vol, sem0, sem1):
    wid = lax.axis_index("s") * NC + lax.axis_index("c")
    z_lo = wid * SLAB
    z_hi = z_lo + SLAB
    zeros16 = jnp.zeros((LANES,), jnp.float32)

    def _zero(i, _):
        vol[pl.ds(i * LANES, LANES)] = zeros16
        return ()
    lax.fori_loop(0, (SLAB * H * W) // LANES, _zero, ())

    pltpu.sync_copy(ranges_hbm, rbuf)
    rv = rbuf[pl.ds(wid * 16, 16)]
    start = rv[0]
    end = rv[1]
    nch = (end - start + (CHUNK - 1)) // CHUNK
    PRE = CHUNK * NROW
    TBL = CHUNK * 128
    sems = (sem0, sem1)

    def start_dmas(ci, slot):
        gbase = start + ci * CHUNK
        sem = sems[slot]
        pltpu.async_copy(params_hbm.at[pl.ds(gbase * NROW, PRE)],
                         pbuf.at[pl.ds(slot * PRE, PRE)], sem)
        pltpu.async_copy(qy_hbm.at[pl.ds(gbase * 128, TBL)],
                         qbuf.at[pl.ds(slot * TBL, TBL)], sem)
        pltpu.async_copy(gy_hbm.at[pl.ds(gbase * 128, TBL)],
                         gbuf.at[pl.ds(slot * TBL, TBL)], sem)
        pltpu.async_copy(iyx_hbm.at[pl.ds(gbase * 128, TBL)],
                         ibuf.at[pl.ds(slot * TBL, TBL)], sem)

    def wait_dmas(slot):
        sem = sems[slot]
        pltpu.make_async_copy(params_hbm.at[pl.ds(0, PRE)],
                              pbuf.at[pl.ds(slot * PRE, PRE)], sem).wait()
        pltpu.make_async_copy(qy_hbm.at[pl.ds(0, TBL)],
                              qbuf.at[pl.ds(slot * TBL, TBL)], sem).wait()
        pltpu.make_async_copy(gy_hbm.at[pl.ds(0, TBL)],
                              gbuf.at[pl.ds(slot * TBL, TBL)], sem).wait()
        pltpu.make_async_copy(iyx_hbm.at[pl.ds(0, TBL)],
                              ibuf.at[pl.ds(slot * TBL, TBL)], sem).wait()

    inv = jnp.float32(INV64)

    def process(ci, slot):
        gbase = start + ci * CHUNK
        cnt = jnp.minimum(CHUNK, end - gbase)
        po = slot * PRE
        to = slot * TBL

        def per_gaussian(gl, _):
            pv = pbuf[pl.ds(po + gl * NROW, NROW)]
            c0s = pv[0]
            nm00 = pv[3]
            r0i = pv[10].astype(jnp.int32)
            thv = jnp.full((LANES,), pv[13] + jnp.float32(-4.5),
                           jnp.float32)
            qv = [qbuf[pl.ds(to + gl * 128 + j * LANES, LANES)]
                  for j in range(NJ)]
            gv = [gbuf[pl.ds(to + gl * 128 + j * LANES, LANES)]
                  for j in range(NJ)]
            iv = [ibuf[pl.ds(to + gl * 128 + j * LANES, LANES)]
                  for j in range(NJ)]
            rzi = pv[14].astype(jnp.int32)
            lo = jnp.maximum(r0i - rzi, z_lo)
            hi = jnp.minimum(r0i + rzi + 1, z_hi)

            def per_plane(z, _):
                d0s = (z.astype(jnp.float32) - c0s) * inv
                t0s = nm00 * d0s * d0s
                d0 = jnp.full((LANES,), d0s, jnp.float32)
                t0 = jnp.full((LANES,), t0s, jnp.float32)
                zoff = jnp.full((LANES,), (z - z_lo) * (H * W), jnp.int32)
                for j in range(NJ):
                    e = t0 + qv[j] + d0 * gv[j]
                    w = jnp.exp(e)
                    msk = e >= thv
                    plsc.addupdate_scatter(vol, [zoff + iv[j]], w, mask=msk)
                return ()

            lax.fori_loop(lo, hi, per_plane, ())
            return ()

        lax.fori_loop(0, cnt, per_gaussian, ())

    @pl.when(nch > 0)
    def _():
        start_dmas(0, 0)

    @pl.when(nch > 1)
    def _():
        start_dmas(1, 1)

    def pair_body(pi, _):
        for slot in range(2):
            ci = pi * 2 + slot

            @pl.when(ci < nch)
            def _():
                wait_dmas(slot)
                process(ci, slot)

                @pl.when(ci + 2 < nch)
                def _():
                    start_dmas(ci + 2, slot)
        return ()

    lax.fori_loop(0, (nch + 1) // 2, pair_body, ())
    pltpu.sync_copy(vol, out_hbm.at[pl.ds(wid * (SLAB * H * W), SLAB * H * W)])


@jax.jit
def _voxelize(positions, scales, rotations, density):
    n = positions.shape[0]
    npad = ((n + 127) // 128) * 128
    rows = npad // 128
    npad2 = ((npad + CHUNK + TBLK - 1) // TBLK) * TBLK

    def pad(x, fill):
        return jnp.concatenate(
            [x, jnp.full((npad - n,) + x.shape[1:], fill, x.dtype)], axis=0)

    p = pad(positions, 1.0e5)
    s = pad(scales, 1.0)
    q = pad(rotations, 1.0)
    dn = pad(density, 0.0)

    def col(x, i):
        return x[:, i].reshape(rows, 128)

    ins = [col(p, 0), col(p, 1), col(p, 2),
           col(s, 0), col(s, 1), col(s, 2),
           col(q, 0), col(q, 1), col(q, 2), col(q, 3),
           dn.reshape(rows, 128)]

    params = pl.pallas_call(
        _preprocess_body,
        out_shape=jax.ShapeDtypeStruct((NROW, rows, 128), jnp.float32),
    )(*ins)
    params = params.reshape(NROW, npad)

    order = jnp.argsort(params[10])
    ps = params[:, order]
    ps = jnp.pad(ps, ((0, 0), (0, npad2 - npad)), constant_values=1.0e9)
    r0s = ps[10]
    zlos = (jnp.arange(NW, dtype=jnp.int32) * SLAB).astype(jnp.float32)
    starts = jnp.searchsorted(r0s, zlos - RADIUS, side="left")
    ends = jnp.searchsorted(r0s, zlos + SLAB - 1 + RADIUS, side="right")
    ranges = jnp.zeros((NW, 16), jnp.int32)
    ranges = ranges.at[:, 0].set(starts.astype(jnp.int32))
    ranges = ranges.at[:, 1].set(ends.astype(jnp.int32))

    qy, gy, iyx = pl.pallas_call(
        _tables_body,
        grid=(npad2 // TBLK,),
        in_specs=[pl.BlockSpec((NROW, TBLK), lambda i: (0, i))],
        out_specs=[pl.BlockSpec((TBLK, 128), lambda i: (i, 0)),
                   pl.BlockSpec((TBLK, 128), lambda i: (i, 0)),
                   pl.BlockSpec((TBLK, 128), lambda i: (i, 0))],
        out_shape=[jax.ShapeDtypeStruct((npad2, 128), jnp.float32),
                   jax.ShapeDtypeStruct((npad2, 128), jnp.float32),
                   jax.ShapeDtypeStruct((npad2, 128), jnp.int32)],
    )(ps)

    mesh = plsc.VectorSubcoreMesh(
        core_axis_name="c", subcore_axis_name="s", num_cores=NC,
        num_subcores=NS)
    vol_flat = pl.kernel(
        _sc_body,
        out_type=jax.ShapeDtypeStruct((D * H * W,), jnp.float32),
        mesh=mesh,
        compiler_params=pltpu.CompilerParams(needs_layout_passes=False),
        scratch_types=[
            pltpu.VMEM((NW * 16,), jnp.int32),
            pltpu.VMEM((2 * CHUNK * NROW,), jnp.float32),
            pltpu.VMEM((2 * CHUNK * 128,), jnp.float32),
            pltpu.VMEM((2 * CHUNK * 128,), jnp.float32),
            pltpu.VMEM((2 * CHUNK * 128,), jnp.int32),
            pltpu.VMEM((SLAB * H * W,), jnp.float32),
            pltpu.SemaphoreType.DMA,
            pltpu.SemaphoreType.DMA,
        ],
    )(ranges.reshape(-1), ps.T.reshape(-1), qy.reshape(-1),
      gy.reshape(-1), iyx.reshape(-1))
    return vol_flat.reshape(D, H, W)


def kernel(positions, scales, rotations, density, chunk_size=2048):
    del chunk_size
    return _voxelize(positions, scales, rotations, density)
